# baseline (speedup 1.0000x reference)
import functools

import jax
import jax.numpy as jnp
from jax import lax
from jax.experimental import pallas as pl
from jax.experimental.pallas import tpu as pltpu
from jax.experimental.pallas import tpu_sc as plsc

EPS = 1e-05
_N = 65536
_D = 64
_NW = 32
_RAYS_PER_W = _N // _NW
_CHUNK = 512
_NCHUNK = _RAYS_PER_W // _CHUNK


def _sc_body(w_hbm, o_hbm, buf0, buf1, sem_i0, sem_i1, sem_o0, sem_o1):
    cid = lax.axis_index("c")
    sid = lax.axis_index("s")
    wid = sid * 2 + cid
    base = wid * _RAYS_PER_W

    bufs = (buf0, buf1)
    in_sems = (sem_i0, sem_i1)
    out_sems = (sem_o0, sem_o1)

    def start_in(g):
        return pltpu.async_copy(
            w_hbm.at[:, pl.ds(base + g * _CHUNK, _CHUNK)], bufs[g % 2], in_sems[g % 2]
        )

    def start_out(g):
        return pltpu.async_copy(
            bufs[g % 2], o_hbm.at[:, pl.ds(base + g * _CHUNK, _CHUNK)], out_sems[g % 2]
        )

    def compute(buf):
        def col_group(j, carry):
            sl = pl.ds(j * 16, 16)
            s = buf[0, sl]
            for c in range(1, _D):
                s = s + buf[c, sl]
            pad = jnp.maximum(EPS - s, 0.0)
            inv = 1.0 / (s + pad)
            a = pad * (1.0 / _D)
            for c in range(_D):
                buf[c, sl] = (buf[c, sl] + a) * inv
            return carry

        lax.fori_loop(0, _CHUNK // 16, col_group, 0)

    in_dma = [None] * _NCHUNK
    out_dma = [None] * _NCHUNK
    in_dma[0] = start_in(0)
    for g in range(_NCHUNK):
        if g + 1 < _NCHUNK:
            if g >= 1:
                out_dma[g - 1].wait()
            in_dma[g + 1] = start_in(g + 1)
        in_dma[g].wait()
        compute(bufs[g % 2])
        out_dma[g] = start_out(g)
    out_dma[_NCHUNK - 2].wait()
    out_dma[_NCHUNK - 1].wait()


@functools.cache
def _sc_pdf():
    mesh = plsc.VectorSubcoreMesh(core_axis_name="c", subcore_axis_name="s")
    return pl.kernel(
        _sc_body,
        out_type=jax.ShapeDtypeStruct((_D, _N), jnp.float32),
        mesh=mesh,
        scratch_types=[
            pltpu.VMEM((_D, _CHUNK), jnp.float32),
            pltpu.VMEM((_D, _CHUNK), jnp.float32),
            pltpu.SemaphoreType.DMA,
            pltpu.SemaphoreType.DMA,
            pltpu.SemaphoreType.DMA,
            pltpu.SemaphoreType.DMA,
        ],
    )


def kernel(weights, stratified):
    wt = weights.T
    out_t = _sc_pdf()(wt)
    return out_t.T
